# baseline (device time: 21479 ns/iter reference)
import jax
import jax.numpy as jnp
from jax import lax
from jax.experimental import pallas as pl
from jax.experimental.pallas import tpu as pltpu

N_DEV = 4
BLK = 256
W = 128


def kernel(x):
    m, n = x.shape
    n_blk = m // BLK
    nc = n // W

    def body(x_hbm, out_hbm, xv, ov, send_buf, comm_ref,
             in_sems, out_sems, send_sems, recv_sems):
        my_pos = lax.axis_index("i")

        barrier_sem = pltpu.get_barrier_semaphore()
        for d in range(1, N_DEV):
            tgt = lax.rem(my_pos + d, N_DEV)
            pl.semaphore_signal(
                barrier_sem, inc=1,
                device_id=(tgt,), device_id_type=pl.DeviceIdType.MESH,
            )
        pl.semaphore_wait(barrier_sem, N_DEV - 1)

        in_copies = []
        for c in range(nc):
            cp = pltpu.make_async_copy(
                x_hbm.at[:, pl.ds(c * W, W)],
                xv.at[:, pl.ds(c * W, W)],
                in_sems.at[c],
            )
            cp.start()
            in_copies.append(cp)

        row = lax.broadcasted_iota(jnp.int32, (BLK, BLK), 0)
        col = lax.broadcasted_iota(jnp.int32, (BLK, BLK), 1)
        tril = (col <= row).astype(jnp.float32)

        send_rdmas = []
        for c in range(nc):
            in_copies[c].wait()
            carry = jnp.zeros((1, W), jnp.float32)
            for b in range(n_blk):
                blk = xv[pl.ds(b * BLK, BLK), pl.ds(c * W, W)]
                cs = lax.dot_general(
                    tril, blk,
                    dimension_numbers=(((1,), (0,)), ((), ())),
                    preferred_element_type=jnp.float32,
                ) + carry
                ov[pl.ds(b * BLK, BLK), pl.ds(c * W, W)] = cs
                carry = cs[BLK - 1 : BLK, :]
            send_buf[c] = carry
            chunk_rdmas = []
            for d in range(1, N_DEV):
                tgt = lax.rem(my_pos + d, N_DEV)
                rdma = pltpu.make_async_remote_copy(
                    src_ref=send_buf.at[c],
                    dst_ref=comm_ref.at[d - 1, c],
                    send_sem=send_sems.at[d - 1, c],
                    recv_sem=recv_sems.at[d - 1, c],
                    device_id=(tgt,),
                    device_id_type=pl.DeviceIdType.MESH,
                )
                rdma.start()
                chunk_rdmas.append(rdma)
            send_rdmas.append(chunk_rdmas)

        out_copies = []
        for c in range(nc):
            offset = jnp.zeros((1, W), jnp.float32)
            for d in range(1, N_DEV):
                send_rdmas[c][d - 1].wait_recv()
                offset = offset + jnp.where(
                    d <= my_pos, comm_ref[d - 1, c], 0.0
                )
            ov[:, pl.ds(c * W, W)] = ov[:, pl.ds(c * W, W)] + offset
            cp = pltpu.make_async_copy(
                ov.at[:, pl.ds(c * W, W)],
                out_hbm.at[:, pl.ds(c * W, W)],
                out_sems.at[c],
            )
            cp.start()
            out_copies.append(cp)

        for c in range(nc):
            for d in range(1, N_DEV):
                send_rdmas[c][d - 1].wait_send()
            out_copies[c].wait()

    return pl.pallas_call(
        body,
        out_shape=jax.ShapeDtypeStruct((m, n), jnp.float32),
        in_specs=[pl.BlockSpec(memory_space=pl.ANY)],
        out_specs=pl.BlockSpec(memory_space=pl.ANY),
        scratch_shapes=[
            pltpu.VMEM((m, n), jnp.float32),
            pltpu.VMEM((m, n), jnp.float32),
            pltpu.VMEM((nc, 1, W), jnp.float32),
            pltpu.VMEM((N_DEV - 1, nc, 1, W), jnp.float32),
            pltpu.SemaphoreType.DMA((nc,)),
            pltpu.SemaphoreType.DMA((nc,)),
            pltpu.SemaphoreType.DMA((N_DEV - 1, nc)),
            pltpu.SemaphoreType.DMA((N_DEV - 1, nc)),
        ],
        compiler_params=pltpu.CompilerParams(collective_id=0),
    )(x)


# device time: 13091 ns/iter; 1.6407x vs baseline; 1.6407x over previous
import functools

import jax
import jax.numpy as jnp
from jax import lax
from jax.experimental import pallas as pl
from jax.experimental.pallas import tpu as pltpu

N_DEV = 4
BLK = 128
N_CHUNK = 2048 // BLK
WB = 128
N_WB = 2048 // WB


def kernel(x):
    m, n = x.shape

    def body(x_ref, out_hbm, ov, send_buf, comm_ref, offset_ref,
             out_sems, send_sems, recv_sems):
        my_pos = lax.axis_index("i")

        barrier_sem = pltpu.get_barrier_semaphore()
        for d in range(1, N_DEV):
            tgt = lax.rem(my_pos + d, N_DEV)
            pl.semaphore_signal(
                barrier_sem, inc=1,
                device_id=(tgt,), device_id_type=pl.DeviceIdType.MESH,
            )

        send_buf[:, :] = jnp.sum(x_ref[:, :], axis=0, keepdims=True)

        pl.semaphore_wait(barrier_sem, N_DEV - 1)

        rdmas = []
        for d in range(1, N_DEV):
            tgt = lax.rem(my_pos + d, N_DEV)
            rdma = pltpu.make_async_remote_copy(
                src_ref=send_buf,
                dst_ref=comm_ref.at[d - 1],
                send_sem=send_sems.at[d - 1],
                recv_sem=recv_sems.at[d - 1],
                device_id=(tgt,),
                device_id_type=pl.DeviceIdType.MESH,
            )
            rdma.start()
            rdmas.append(rdma)

        row = lax.broadcasted_iota(jnp.int32, (BLK, BLK), 0)
        col = lax.broadcasted_iota(jnp.int32, (BLK, BLK), 1)
        tril = (col <= row).astype(jnp.float32)
        carry = jnp.zeros((1, n), jnp.float32)
        for r in range(N_CHUNK):
            cs = lax.dot_general(
                tril, x_ref[pl.ds(r * BLK, BLK), :],
                dimension_numbers=(((1,), (0,)), ((), ())),
                preferred_element_type=jnp.float32,
            ) + carry
            ov[pl.ds(r * BLK, BLK), :] = cs
            carry = cs[BLK - 1 : BLK, :]

        offset_ref[:, :] = jnp.zeros((1, n), jnp.float32)
        for d in range(1, N_DEV):
            @pl.when(d <= my_pos)
            def _(d=d):
                rdmas[d - 1].wait_recv()
                offset_ref[:, :] = offset_ref[:, :] + comm_ref[d - 1, :, :]
        offset = offset_ref[:, :]

        out_copies = []
        for r in range(N_WB):
            ov[pl.ds(r * WB, WB), :] = ov[pl.ds(r * WB, WB), :] + offset
            cp = pltpu.make_async_copy(
                ov.at[pl.ds(r * WB, WB), :],
                out_hbm.at[pl.ds(r * WB, WB), :],
                out_sems.at[r],
            )
            cp.start()
            out_copies.append(cp)

        for d in range(1, N_DEV):
            @pl.when(d > my_pos)
            def _(d=d):
                rdmas[d - 1].wait_recv()
        for d in range(1, N_DEV):
            rdmas[d - 1].wait_send()
        for r in range(N_WB):
            out_copies[r].wait()

    return pl.pallas_call(
        body,
        out_shape=jax.ShapeDtypeStruct((m, n), jnp.float32),
        in_specs=[pl.BlockSpec(memory_space=pltpu.VMEM)],
        out_specs=pl.BlockSpec(memory_space=pl.ANY),
        scratch_shapes=[
            pltpu.VMEM((m, n), jnp.float32),
            pltpu.VMEM((1, n), jnp.float32),
            pltpu.VMEM((N_DEV - 1, 1, n), jnp.float32),
            pltpu.VMEM((1, n), jnp.float32),
            pltpu.SemaphoreType.DMA((N_WB,)),
            pltpu.SemaphoreType.DMA((N_DEV - 1,)),
            pltpu.SemaphoreType.DMA((N_DEV - 1,)),
        ],
        compiler_params=pltpu.CompilerParams(collective_id=0),
    )(x)
